# baseline (speedup 1.0000x reference)
import jax
import jax.numpy as jnp
from jax import lax
from jax.experimental import pallas as pl
from jax.experimental.pallas import tpu as pltpu

VOCAB_N = 1000000
DIM = 64
CTX = 200
BLKV = 16384
NB = pl.cdiv(VOCAB_N, BLKV)
INV_CTX = 1.0 / CTX


def _cbow_body(idx_ref, tbl_ref, wt_ref, b_ref, out_ref,
               cols, vscr, lbuf, m_ref, s_ref, lse_ref, sem):
    i = pl.program_id(0)

    @pl.when(i == 0)
    def _gather_and_mean():
        m_ref[0] = -jnp.inf
        s_ref[0] = 0.0
        cps = []
        for t in range(CTX):
            c_al = pl.multiple_of(
                lax.shift_left(lax.shift_right_logical(idx_ref[t], 7), 7),
                128)
            cp = pltpu.make_async_copy(
                tbl_ref.at[:, pl.ds(c_al, 128)], cols.at[t], sem)
            cp.start()
            cps.append(cp)
        for cp in cps:
            cp.wait()
        lane = lax.broadcasted_iota(jnp.int32, (DIM, 128), 1)
        accs = [jnp.zeros((DIM, 128), jnp.float32) for _ in range(4)]
        for t in range(CTX):
            p_t = jnp.bitwise_and(idx_ref[t], 127)
            accs[t % 4] = accs[t % 4] + jnp.where(lane == p_t, cols[t], 0.0)
        acc = (accs[0] + accs[1]) + (accs[2] + accs[3])
        vscr[:, 0:1] = jnp.sum(acc, axis=1, keepdims=True) * INV_CTX

    @pl.when(i < NB)
    def _phase1():
        v = vscr[:, 0:1]
        xb = lax.dot_general(
            v, wt_ref[...], (((0,), (0,)), ((), ())),
            preferred_element_type=jnp.float32,
        )
        xb = xb + b_ref[...]
        lbuf[pl.ds(i, 1), :] = xb

        col = lax.broadcasted_iota(jnp.int32, (1, BLKV), 1) + i * BLKV
        xm = jnp.where(col < VOCAB_N, xb, -jnp.inf)
        bm = jnp.max(xm)
        m_old = m_ref[0]
        m_new = jnp.maximum(m_old, bm)
        s_ref[0] = (s_ref[0] * jnp.exp(m_old - m_new)
                    + jnp.sum(jnp.exp(xm - m_new)))
        m_ref[0] = m_new

        @pl.when(i == NB - 1)
        def _finish():
            lse_ref[0] = m_ref[0] + jnp.log(s_ref[0])

    @pl.when(i >= NB)
    def _phase2():
        j = i - NB
        out_ref[...] = lbuf[pl.ds(j, 1), :] - lse_ref[0]


_cbow_call = pl.pallas_call(
    _cbow_body,
    grid_spec=pltpu.PrefetchScalarGridSpec(
        num_scalar_prefetch=1,
        grid=(2 * NB,),
        in_specs=[
            pl.BlockSpec(memory_space=pl.ANY),
            pl.BlockSpec(
                (DIM, BLKV),
                lambda i, idx_ref: (0, jnp.minimum(i, NB - 1))),
            pl.BlockSpec(
                (1, BLKV),
                lambda i, idx_ref: (0, jnp.minimum(i, NB - 1))),
        ],
        out_specs=[
            pl.BlockSpec(
                (1, BLKV),
                lambda i, idx_ref: (0, jnp.maximum(i - NB, 0))),
        ],
        scratch_shapes=[
            pltpu.VMEM((CTX, DIM, 128), jnp.float32),
            pltpu.VMEM((DIM, 128), jnp.float32),
            pltpu.VMEM((NB, BLKV), jnp.float32),
            pltpu.SMEM((1,), jnp.float32),
            pltpu.SMEM((1,), jnp.float32),
            pltpu.SMEM((1,), jnp.float32),
            pltpu.SemaphoreType.DMA,
        ],
    ),
    out_shape=[
        jax.ShapeDtypeStruct((1, VOCAB_N), jnp.float32),
    ],
    compiler_params=pltpu.CompilerParams(
        dimension_semantics=("arbitrary",),
    ),
)


def kernel(inputs, table, W, b):
    idx = inputs.astype(jnp.int32)
    (out,) = _cbow_call(idx, table.T, W.T, b.reshape(1, VOCAB_N))
    return out


# speedup vs baseline: 1.2908x; 1.2908x over previous
import jax
import jax.numpy as jnp
from jax import lax
from jax.experimental import pallas as pl
from jax.experimental.pallas import tpu as pltpu

VOCAB_N = 1000000
DIM = 64
CTX = 200
BLKV = 32768
NB = pl.cdiv(VOCAB_N, BLKV)
SBLK = 131072
NSUB = pl.cdiv(VOCAB_N, SBLK)
INV_CTX = 1.0 / CTX


def _cbow_body(idx_ref, tbl_ref, wt_ref, b_ref, out_ref, lse_ref,
               cols, vscr, m_ref, s_ref, sem):
    i = pl.program_id(0)

    @pl.when(i == 0)
    def _gather_and_mean():
        m_ref[0] = -jnp.inf
        s_ref[0] = 0.0
        cps = []
        for t in range(CTX):
            c_al = pl.multiple_of(
                lax.shift_left(lax.shift_right_logical(idx_ref[t], 7), 7),
                128)
            cp = pltpu.make_async_copy(
                tbl_ref.at[:, pl.ds(c_al, 128)], cols.at[t], sem)
            cp.start()
            cps.append(cp)
        for cp in cps:
            cp.wait()
        lane = lax.broadcasted_iota(jnp.int32, (DIM, 128), 1)
        accs = [jnp.zeros((DIM, 128), jnp.float32) for _ in range(4)]
        for t in range(CTX):
            p_t = jnp.bitwise_and(idx_ref[t], 127)
            accs[t % 4] = accs[t % 4] + jnp.where(lane == p_t, cols[t], 0.0)
        acc = (accs[0] + accs[1]) + (accs[2] + accs[3])
        vscr[:, 0:1] = jnp.sum(acc, axis=1, keepdims=True) * INV_CTX

    v = vscr[:, 0:1]
    xb = lax.dot_general(
        v, wt_ref[...], (((0,), (0,)), ((), ())),
        preferred_element_type=jnp.float32,
    )
    xb = xb + b_ref[...]
    out_ref[...] = xb

    col = lax.broadcasted_iota(jnp.int32, (1, BLKV), 1) + i * BLKV
    xm = jnp.where(col < VOCAB_N, xb, -jnp.inf)
    bm = jnp.max(xm)
    m_old = m_ref[0]
    m_new = jnp.maximum(m_old, bm)
    s_ref[0] = s_ref[0] * jnp.exp(m_old - m_new) + jnp.sum(jnp.exp(xm - m_new))
    m_ref[0] = m_new

    @pl.when(i == NB - 1)
    def _finish():
        lse_ref[...] = jnp.full((1, 1), m_ref[0] + jnp.log(s_ref[0]),
                                jnp.float32)


_cbow_call = pl.pallas_call(
    _cbow_body,
    grid_spec=pltpu.PrefetchScalarGridSpec(
        num_scalar_prefetch=1,
        grid=(NB,),
        in_specs=[
            pl.BlockSpec(memory_space=pl.ANY),
            pl.BlockSpec((DIM, BLKV), lambda i, idx_ref: (0, i)),
            pl.BlockSpec((1, BLKV), lambda i, idx_ref: (0, i)),
        ],
        out_specs=[
            pl.BlockSpec((1, BLKV), lambda i, idx_ref: (0, i)),
            pl.BlockSpec((1, 1), lambda i, idx_ref: (0, 0)),
        ],
        scratch_shapes=[
            pltpu.VMEM((CTX, DIM, 128), jnp.float32),
            pltpu.VMEM((DIM, 128), jnp.float32),
            pltpu.SMEM((1,), jnp.float32),
            pltpu.SMEM((1,), jnp.float32),
            pltpu.SemaphoreType.DMA,
        ],
    ),
    out_shape=[
        jax.ShapeDtypeStruct((1, VOCAB_N), jnp.float32),
        jax.ShapeDtypeStruct((1, 1), jnp.float32),
    ],
    compiler_params=pltpu.CompilerParams(
        dimension_semantics=("arbitrary",),
    ),
)


def _sub_body(x_ref, lse_ref, o_ref):
    o_ref[...] = x_ref[...] - lse_ref[0, 0]


_sub_call = pl.pallas_call(
    _sub_body,
    grid=(NSUB,),
    in_specs=[
        pl.BlockSpec((1, SBLK), lambda i: (0, i)),
        pl.BlockSpec(memory_space=pltpu.SMEM),
    ],
    out_specs=pl.BlockSpec((1, SBLK), lambda i: (0, i)),
    out_shape=jax.ShapeDtypeStruct((1, VOCAB_N), jnp.float32),
    compiler_params=pltpu.CompilerParams(
        dimension_semantics=("arbitrary",),
    ),
)


def kernel(inputs, table, W, b):
    idx = inputs.astype(jnp.int32)
    logits, lse = _cbow_call(idx, table.T, W.T, b.reshape(1, VOCAB_N))
    return _sub_call(logits, lse)
